# Initial kernel scaffold; baseline (speedup 1.0000x reference)
import functools

import jax
import jax.numpy as jnp
from jax import lax
from jax.experimental import pallas as pl
from jax.experimental.pallas import tpu as pltpu
from jax.experimental.pallas import tpu_sc as plsc

VOCAB = 100000
EMBED_DIM = 64
B = 16384
L = 50
TOT = B * L

_info = plsc.get_sparse_core_info()
NC, NS = _info.num_cores, _info.num_subcores
NW = NC * NS
PER_W = TOT // NW
CHUNK = 1024
NCHUNK = PER_W // CHUNK

_mesh = plsc.VectorSubcoreMesh(core_axis_name="c", subcore_axis_name="s")


@functools.partial(
    pl.kernel,
    mesh=_mesh,
    out_type=jax.ShapeDtypeStruct((TOT, EMBED_DIM), jnp.float32),
    scratch_types=[
        pltpu.VMEM((CHUNK,), jnp.int32),
        pltpu.VMEM((CHUNK, EMBED_DIM), jnp.float32),
        pltpu.SemaphoreType.DMA,
    ],
)
def _gather_sc(idx_hbm, table_hbm, out_hbm, idx_v, rows_v, sem):
    wid = lax.axis_index("s") * NC + lax.axis_index("c")
    base = wid * PER_W

    def body(i, carry):
        off = pl.multiple_of(base + i * CHUNK, CHUNK)
        pltpu.sync_copy(idx_hbm.at[pl.ds(off, CHUNK)], idx_v)
        pltpu.async_copy(table_hbm.at[idx_v], rows_v, sem).wait()
        pltpu.sync_copy(rows_v, out_hbm.at[pl.ds(off, CHUNK)])
        return carry

    lax.fori_loop(0, NCHUNK, body, 0)


def kernel(indices, table):
    flat = indices.reshape(-1).astype(jnp.int32)
    out = _gather_sc(flat, table)
    return out.reshape(B, L, EMBED_DIM)


# speedup vs baseline: 6.0536x; 6.0536x over previous
import functools

import jax
import jax.numpy as jnp
from jax import lax
from jax.experimental import pallas as pl
from jax.experimental.pallas import tpu as pltpu
from jax.experimental.pallas import tpu_sc as plsc

VOCAB = 100000
EMBED_DIM = 64
B = 16384
L = 50
TOT = B * L

_info = plsc.get_sparse_core_info()
NC, NS = _info.num_cores, _info.num_subcores
NW = NC * NS
PER_W = TOT // NW
CHUNK = 1024
NCHUNK = PER_W // CHUNK

_mesh = plsc.VectorSubcoreMesh(core_axis_name="c", subcore_axis_name="s")


@functools.partial(
    pl.kernel,
    mesh=_mesh,
    out_type=jax.ShapeDtypeStruct((TOT, EMBED_DIM), jnp.float32),
    scratch_types=[
        pltpu.VMEM((CHUNK,), jnp.int32),
        pltpu.VMEM((CHUNK, EMBED_DIM), jnp.float32),
        pltpu.SemaphoreType.DMA,
    ],
    compiler_params=pltpu.CompilerParams(use_tc_tiling_on_sc=False),
)
def _gather_sc(idx_hbm, table_hbm, out_hbm, idx_v, rows_v, sem):
    wid = lax.axis_index("s") * NC + lax.axis_index("c")
    base = wid * PER_W

    def body(i, carry):
        off = pl.multiple_of(base + i * CHUNK, CHUNK)
        pltpu.sync_copy(idx_hbm.at[pl.ds(off, CHUNK)], idx_v)
        pltpu.async_copy(table_hbm.at[idx_v], rows_v, sem).wait()
        pltpu.sync_copy(rows_v, out_hbm.at[pl.ds(off, CHUNK)])
        return carry

    lax.fori_loop(0, NCHUNK, body, 0)


def kernel(indices, table):
    flat = indices.reshape(-1).astype(jnp.int32)
    out = _gather_sc(flat, table)
    return out.reshape(B, L, EMBED_DIM)


# speedup vs baseline: 6.2700x; 1.0357x over previous
import functools

import jax
import jax.numpy as jnp
from jax import lax
from jax.experimental import pallas as pl
from jax.experimental.pallas import tpu as pltpu
from jax.experimental.pallas import tpu_sc as plsc

VOCAB = 100000
EMBED_DIM = 64
B = 16384
L = 50
TOT = B * L

_info = plsc.get_sparse_core_info()
NC, NS = _info.num_cores, _info.num_subcores
NW = NC * NS
PER_W = TOT // NW
CHUNK = 640
NCHUNK = PER_W // CHUNK

_mesh = plsc.VectorSubcoreMesh(core_axis_name="c", subcore_axis_name="s")


@functools.partial(
    pl.kernel,
    mesh=_mesh,
    out_type=jax.ShapeDtypeStruct((TOT, EMBED_DIM), jnp.float32),
    scratch_types=[
        pltpu.VMEM((CHUNK,), jnp.int32),
        pltpu.VMEM((CHUNK,), jnp.int32),
        pltpu.VMEM((CHUNK, EMBED_DIM), jnp.float32),
        pltpu.VMEM((CHUNK, EMBED_DIM), jnp.float32),
        pltpu.SemaphoreType.DMA,
        pltpu.SemaphoreType.DMA,
        pltpu.SemaphoreType.DMA,
        pltpu.SemaphoreType.DMA,
        pltpu.SemaphoreType.DMA,
        pltpu.SemaphoreType.DMA,
    ],
    compiler_params=pltpu.CompilerParams(use_tc_tiling_on_sc=False),
)
def _gather_sc(
    idx_hbm, table_hbm, out_hbm,
    idx0, idx1, rows0, rows1,
    is0, is1, gs0, gs1, ss0, ss1,
):
    wid = lax.axis_index("s") * NC + lax.axis_index("c")
    base = wid * PER_W
    idxb = (idx0, idx1)
    rowsb = (rows0, rows1)
    isem = (is0, is1)
    gsem = (gs0, gs1)
    ssem = (ss0, ss1)

    def off(i):
        return pl.multiple_of(base + i * CHUNK, CHUNK)

    def i_start(i, b):
        pltpu.async_copy(idx_hbm.at[pl.ds(off(i), CHUNK)], idxb[b], isem[b])

    def i_wait(b):
        pltpu.make_async_copy(
            idx_hbm.at[pl.ds(base, CHUNK)], idxb[b], isem[b]
        ).wait()

    def g_start(b):
        pltpu.async_copy(table_hbm.at[idxb[b]], rowsb[b], gsem[b])

    def g_wait(b):
        pltpu.make_async_copy(table_hbm.at[idxb[b]], rowsb[b], gsem[b]).wait()

    def s_start(i, b):
        pltpu.async_copy(rowsb[b], out_hbm.at[pl.ds(off(i), CHUNK)], ssem[b])

    def s_wait(b):
        pltpu.make_async_copy(
            rowsb[b], out_hbm.at[pl.ds(base, CHUNK)], ssem[b]
        ).wait()

    i_start(0, 0)
    i_start(1, 1)
    i_wait(0)
    g_start(0)
    g_wait(0)
    i_start(2, 0)
    s_start(0, 0)
    i_wait(1)
    g_start(1)

    @pl.loop(2, NCHUNK, step=2)
    def _(outer):
        for d in range(2):
            i = outer + d
            b = d
            nb = 1 - b
            s_wait(b)
            i_wait(b)
            g_start(b)
            g_wait(nb)
            @pl.when(i + 1 < NCHUNK)
            def _prefetch():
                i_start(i + 1, nb)
            s_start(i - 1, nb)

    g_wait(1)
    s_start(NCHUNK - 1, 1)
    s_wait(0)
    s_wait(1)


def kernel(indices, table):
    flat = indices.reshape(-1).astype(jnp.int32)
    out = _gather_sc(flat, table)
    return out.reshape(B, L, EMBED_DIM)
